# initial kernel scaffold (unmeasured)
import jax
import jax.numpy as jnp
from jax import lax
from jax.experimental import pallas as pl
from jax.experimental.pallas import tpu as pltpu

N_DEV = 32
B, SQ, DM = 2, 256, 768
HQ_SH, DH = 4, 64
DSH = HQ_SH * DH
ROWS = B * SQ
CHUNK = ROWS // N_DEV


def kernel(x, Wq, Wk, Wv, Wo):
    def body(x_ref, wq_ref, wk_ref, wv_ref, wo_ref, out_ref,
             part_ref, recv_ref, red_ref, send1, recv1, send2, recv2):
        my = lax.axis_index("i")

        x2 = x_ref[...].reshape(ROWS, DM).astype(jnp.bfloat16)
        q = jnp.dot(x2, wq_ref[...].astype(jnp.bfloat16),
                    preferred_element_type=jnp.float32)
        k = jnp.dot(x2, wk_ref[...].astype(jnp.bfloat16),
                    preferred_element_type=jnp.float32)
        v = jnp.dot(x2, wv_ref[...].astype(jnp.bfloat16),
                    preferred_element_type=jnp.float32)

        rows = lax.broadcasted_iota(jnp.int32, (ROWS, DSH), 0)
        cols = lax.broadcasted_iota(jnp.int32, (ROWS, DSH), 1)
        pos = (rows % SQ).astype(jnp.float32)
        expo = (2 * ((cols % DH) // 2)).astype(jnp.float32) / DH
        ang = pos * jnp.exp(-expo * jnp.log(10000.0))
        cos_t = jnp.cos(ang)
        sin_t = jnp.sin(ang)
        even = (cols % 2) == 0

        def rot(t):
            zero = jnp.zeros((ROWS, 1), t.dtype)
            tm1 = jnp.concatenate([t[:, 1:], zero], axis=1)
            tp1 = jnp.concatenate([zero, t[:, :-1]], axis=1)
            t_r = jnp.where(even, -tm1, tp1)
            return t * cos_t + t_r * sin_t

        q = rot(q).astype(jnp.bfloat16)
        k = rot(k).astype(jnp.bfloat16)
        v = v.astype(jnp.bfloat16)
        wo = wo_ref[...].astype(jnp.bfloat16)

        for b in range(B):
            rb = slice(b * SQ, (b + 1) * SQ)
            ctx = []
            for h in range(HQ_SH):
                cb = slice(h * DH, (h + 1) * DH)
                qh, kh, vh = q[rb, cb], k[rb, cb], v[rb, cb]
                s = lax.dot_general(
                    qh, kh, (((1,), (1,)), ((), ())),
                    preferred_element_type=jnp.float32) * 0.125
                m = jnp.max(s, axis=-1, keepdims=True)
                e = jnp.exp(s - m)
                w = (e / jnp.sum(e, axis=-1, keepdims=True)).astype(jnp.bfloat16)
                ctx.append(jnp.dot(w, vh, preferred_element_type=jnp.float32))
            ctx_b = jnp.concatenate(ctx, axis=1).astype(jnp.bfloat16)
            part_ref[rb, :] = jnp.dot(ctx_b, wo,
                                      preferred_element_type=jnp.float32)

        d1 = []
        for o in range(1, N_DEV):
            tgt = lax.rem(my + o, N_DEV)
            rd = pltpu.make_async_remote_copy(
                src_ref=part_ref.at[pl.ds(CHUNK * tgt, CHUNK)],
                dst_ref=recv_ref.at[o],
                send_sem=send1.at[o],
                recv_sem=recv1.at[o],
                device_id=(tgt,),
                device_id_type=pl.DeviceIdType.MESH,
            )
            rd.start()
            d1.append(rd)

        acc = part_ref[pl.ds(CHUNK * my, CHUNK), :]
        for o in range(1, N_DEV):
            d1[o - 1].wait_recv()
            acc = acc + recv_ref[o, :, :]
        out_ref[pl.ds(CHUNK * my, CHUNK), :] = acc
        red_ref[...] = acc

        d2 = []
        for o in range(1, N_DEV):
            tgt = lax.rem(my + o, N_DEV)
            rd = pltpu.make_async_remote_copy(
                src_ref=red_ref,
                dst_ref=out_ref.at[pl.ds(CHUNK * my, CHUNK)],
                send_sem=send2.at[o],
                recv_sem=recv2.at[o],
                device_id=(tgt,),
                device_id_type=pl.DeviceIdType.MESH,
            )
            rd.start()
            d2.append(rd)
        for rd in d2:
            rd.wait_recv()
        for rd in d1 + d2:
            rd.wait_send()

    out = pl.pallas_call(
        body,
        out_shape=jax.ShapeDtypeStruct((ROWS, DM), jnp.float32),
        in_specs=[pl.BlockSpec(memory_space=pltpu.VMEM)] * 5,
        out_specs=pl.BlockSpec(memory_space=pltpu.VMEM),
        scratch_shapes=[
            pltpu.VMEM((ROWS, DM), jnp.float32),
            pltpu.VMEM((N_DEV, CHUNK, DM), jnp.float32),
            pltpu.VMEM((CHUNK, DM), jnp.float32),
            pltpu.SemaphoreType.DMA((N_DEV,)),
            pltpu.SemaphoreType.DMA((N_DEV,)),
            pltpu.SemaphoreType.DMA((N_DEV,)),
            pltpu.SemaphoreType.DMA((N_DEV,)),
        ],
        compiler_params=pltpu.CompilerParams(collective_id=0),
    )(x, Wq, Wk, Wv, Wo)
    return out.reshape(B, SQ, DM)


# baseline (device time: 63278 ns/iter reference)
import jax
import jax.numpy as jnp
from jax import lax
from jax.experimental import pallas as pl
from jax.experimental.pallas import tpu as pltpu

N_DEV = 32
B, SQ, DM = 2, 256, 768
HQ_SH, DH = 4, 64
DSH = HQ_SH * DH
ROWS = B * SQ
CHUNK = ROWS // N_DEV


def kernel(x, Wq, Wk, Wv, Wo):
    def body(x_ref, wq_ref, wk_ref, wv_ref, wo_ref, out_ref,
             part_ref, recv_ref, red_ref, send1, recv1, send2, recv2):
        my = lax.axis_index("i")

        x2 = x_ref[...].reshape(ROWS, DM).astype(jnp.bfloat16)
        q = jnp.dot(x2, wq_ref[...].astype(jnp.bfloat16),
                    preferred_element_type=jnp.float32)
        k = jnp.dot(x2, wk_ref[...].astype(jnp.bfloat16),
                    preferred_element_type=jnp.float32)
        v = jnp.dot(x2, wv_ref[...].astype(jnp.bfloat16),
                    preferred_element_type=jnp.float32)

        rows = lax.broadcasted_iota(jnp.int32, (ROWS, DSH), 0)
        cols = lax.broadcasted_iota(jnp.int32, (ROWS, DSH), 1)
        pos = (rows % SQ).astype(jnp.float32)
        expo = (2 * ((cols % DH) // 2)).astype(jnp.float32) / DH
        ang = pos * jnp.exp(-expo * jnp.log(10000.0))
        cos_t = jnp.cos(ang)
        sin_t = jnp.sin(ang)
        even = (cols % 2) == 0

        def rot(t):
            zero = jnp.zeros((ROWS, 1), t.dtype)
            tm1 = jnp.concatenate([t[:, 1:], zero], axis=1)
            tp1 = jnp.concatenate([zero, t[:, :-1]], axis=1)
            t_r = jnp.where(even, -tm1, tp1)
            return t * cos_t + t_r * sin_t

        q = rot(q).astype(jnp.bfloat16)
        k = rot(k).astype(jnp.bfloat16)
        v = v.astype(jnp.bfloat16)
        wo = wo_ref[...].astype(jnp.bfloat16)

        for b in range(B):
            rb = slice(b * SQ, (b + 1) * SQ)
            ctx = []
            for h in range(HQ_SH):
                cb = slice(h * DH, (h + 1) * DH)
                qh, kh, vh = q[rb, cb], k[rb, cb], v[rb, cb]
                s = lax.dot_general(
                    qh, kh, (((1,), (1,)), ((), ())),
                    preferred_element_type=jnp.float32) * 0.125
                m = jnp.max(s, axis=-1, keepdims=True)
                e = jnp.exp(s - m)
                w = (e / jnp.sum(e, axis=-1, keepdims=True)).astype(jnp.bfloat16)
                ctx.append(jnp.dot(w, vh, preferred_element_type=jnp.float32))
            ctx_b = jnp.concatenate(ctx, axis=1).astype(jnp.bfloat16)
            part_ref[rb, :] = jnp.dot(ctx_b, wo,
                                      preferred_element_type=jnp.float32)

        d1 = []
        for o in range(1, N_DEV):
            tgt = lax.rem(my + o, N_DEV)
            rd = pltpu.make_async_remote_copy(
                src_ref=part_ref.at[pl.ds(CHUNK * tgt, CHUNK)],
                dst_ref=recv_ref.at[o],
                send_sem=send1.at[o],
                recv_sem=recv1.at[o],
                device_id=(tgt,),
                device_id_type=pl.DeviceIdType.MESH,
            )
            rd.start()
            d1.append(rd)

        acc = part_ref[pl.ds(CHUNK * my, CHUNK), :]
        for o in range(1, N_DEV):
            d1[o - 1].wait_recv()
            acc = acc + recv_ref[o, :, :]
        out_ref[pl.ds(CHUNK * my, CHUNK), :] = acc
        red_ref[...] = acc

        d2 = []
        for o in range(1, N_DEV):
            tgt = lax.rem(my + o, N_DEV)
            rd = pltpu.make_async_remote_copy(
                src_ref=red_ref,
                dst_ref=out_ref.at[pl.ds(CHUNK * my, CHUNK)],
                send_sem=send2.at[o],
                recv_sem=recv2.at[o],
                device_id=(tgt,),
                device_id_type=pl.DeviceIdType.MESH,
            )
            rd.start()
            d2.append(rd)
        for rd in d2:
            rd.wait_recv()
        for rd in d1 + d2:
            rd.wait_send()

    out = pl.pallas_call(
        body,
        out_shape=jax.ShapeDtypeStruct((ROWS, DM), jnp.float32),
        in_specs=[pl.BlockSpec(memory_space=pltpu.VMEM)] * 5,
        out_specs=pl.BlockSpec(memory_space=pltpu.VMEM),
        scratch_shapes=[
            pltpu.VMEM((ROWS, DM), jnp.float32),
            pltpu.VMEM((N_DEV, CHUNK, DM), jnp.float32),
            pltpu.VMEM((CHUNK, DM), jnp.float32),
            pltpu.SemaphoreType.DMA((N_DEV,)),
            pltpu.SemaphoreType.DMA((N_DEV,)),
            pltpu.SemaphoreType.DMA((N_DEV,)),
            pltpu.SemaphoreType.DMA((N_DEV,)),
        ],
    )(x, Wq, Wk, Wv, Wo)
    return out.reshape(B, SQ, DM)


# device time: 46912 ns/iter; 1.3489x vs baseline; 1.3489x over previous
import jax
import jax.numpy as jnp
from jax import lax
from jax.experimental import pallas as pl
from jax.experimental.pallas import tpu as pltpu

N_DEV = 32
B, SQ, DM = 2, 256, 768
HQ_SH, DH = 4, 64
DSH = HQ_SH * DH
ROWS = B * SQ
CHUNK = ROWS // N_DEV


def kernel(x, Wq, Wk, Wv, Wo):
    def body(x_ref, wq_ref, wk_ref, wv_ref, wo_ref, out_ref,
             part_ref, recv_ref, red_ref, send1, recv1, send2, recv2):
        my = lax.axis_index("i")

        x2 = x_ref[...].reshape(ROWS, DM).astype(jnp.bfloat16)
        q = jnp.dot(x2, wq_ref[...].astype(jnp.bfloat16),
                    preferred_element_type=jnp.float32)
        k = jnp.dot(x2, wk_ref[...].astype(jnp.bfloat16),
                    preferred_element_type=jnp.float32)
        v = jnp.dot(x2, wv_ref[...].astype(jnp.bfloat16),
                    preferred_element_type=jnp.float32)

        rows = lax.broadcasted_iota(jnp.int32, (ROWS, DSH), 0)
        cols = lax.broadcasted_iota(jnp.int32, (ROWS, DSH), 1)
        pos = (rows % SQ).astype(jnp.float32)
        expo = (2 * ((cols % DH) // 2)).astype(jnp.float32) / DH
        ang = pos * jnp.exp(-expo * jnp.log(10000.0))
        cos_t = jnp.cos(ang)
        sin_t = jnp.sin(ang)
        even = (cols % 2) == 0

        def rot(t):
            zero = jnp.zeros((ROWS, 1), t.dtype)
            tm1 = jnp.concatenate([t[:, 1:], zero], axis=1)
            tp1 = jnp.concatenate([zero, t[:, :-1]], axis=1)
            t_r = jnp.where(even, -tm1, tp1)
            return t * cos_t + t_r * sin_t

        q = rot(q).astype(jnp.bfloat16)
        k = rot(k).astype(jnp.bfloat16)
        v = v.astype(jnp.bfloat16)
        wo = wo_ref[...].astype(jnp.bfloat16)

        for b in range(B):
            rb = slice(b * SQ, (b + 1) * SQ)
            ctx = []
            for h in range(HQ_SH):
                cb = slice(h * DH, (h + 1) * DH)
                qh, kh, vh = q[rb, cb], k[rb, cb], v[rb, cb]
                s = lax.dot_general(
                    qh, kh, (((1,), (1,)), ((), ())),
                    preferred_element_type=jnp.float32) * 0.125
                m = jnp.max(s, axis=-1, keepdims=True)
                e = jnp.exp(s - m)
                w = (e / jnp.sum(e, axis=-1, keepdims=True)).astype(jnp.bfloat16)
                ctx.append(jnp.dot(w, vh, preferred_element_type=jnp.float32))
            ctx_b = jnp.concatenate(ctx, axis=1).astype(jnp.bfloat16)
            part_ref[rb, :] = jnp.dot(
                ctx_b, wo, preferred_element_type=jnp.float32
            ).astype(jnp.bfloat16)

        d1 = []
        for o in range(1, N_DEV):
            tgt = lax.rem(my + o, N_DEV)
            rd = pltpu.make_async_remote_copy(
                src_ref=part_ref.at[pl.ds(CHUNK * tgt, CHUNK)],
                dst_ref=recv_ref.at[o],
                send_sem=send1.at[o],
                recv_sem=recv1.at[o],
                device_id=(tgt,),
                device_id_type=pl.DeviceIdType.MESH,
            )
            rd.start()
            d1.append(rd)

        acc = part_ref[pl.ds(CHUNK * my, CHUNK), :].astype(jnp.float32)
        for o in range(1, N_DEV):
            d1[o - 1].wait_recv()
            acc = acc + recv_ref[o, :, :].astype(jnp.float32)
        red = acc.astype(jnp.bfloat16)
        out_ref[pl.ds(CHUNK * my, CHUNK), :] = red
        red_ref[...] = red

        d2 = []
        for o in range(1, N_DEV):
            tgt = lax.rem(my + o, N_DEV)
            rd = pltpu.make_async_remote_copy(
                src_ref=red_ref,
                dst_ref=out_ref.at[pl.ds(CHUNK * my, CHUNK)],
                send_sem=send2.at[o],
                recv_sem=recv2.at[o],
                device_id=(tgt,),
                device_id_type=pl.DeviceIdType.MESH,
            )
            rd.start()
            d2.append(rd)
        for rd in d2:
            rd.wait_recv()
        for rd in d1 + d2:
            rd.wait_send()

    out = pl.pallas_call(
        body,
        out_shape=jax.ShapeDtypeStruct((ROWS, DM), jnp.bfloat16),
        in_specs=[pl.BlockSpec(memory_space=pltpu.VMEM)] * 5,
        out_specs=pl.BlockSpec(memory_space=pltpu.VMEM),
        scratch_shapes=[
            pltpu.VMEM((ROWS, DM), jnp.bfloat16),
            pltpu.VMEM((N_DEV, CHUNK, DM), jnp.bfloat16),
            pltpu.VMEM((CHUNK, DM), jnp.bfloat16),
            pltpu.SemaphoreType.DMA((N_DEV,)),
            pltpu.SemaphoreType.DMA((N_DEV,)),
            pltpu.SemaphoreType.DMA((N_DEV,)),
            pltpu.SemaphoreType.DMA((N_DEV,)),
        ],
    )(x, Wq, Wk, Wv, Wo)
    return out.reshape(B, SQ, DM)
